# Initial kernel scaffold; baseline (speedup 1.0000x reference)
import jax
import jax.numpy as jnp
from jax.experimental import pallas as pl


def kernel(entity_ids, adj_entity, adj_relation, entity_embedding, relation_embedding, W_att, b_att, W_conv, b_conv):
    raise NotImplementedError("write your pallas kernel here")


# speedup vs baseline: 19.3672x; 19.3672x over previous
import functools

import jax
import jax.numpy as jnp
from jax import lax
from jax.experimental import pallas as pl
from jax.experimental.pallas import tpu as pltpu
from jax.experimental.pallas import tpu_sc as plsc

N_ENT = 100000
N_REL = 100
D = 64
K = 16
L = 20

NC = 2
NS = 16
NW = NC * NS

CB = 64
SB = 4
CD = 16
SCH = 32

_SC_PARAMS = pltpu.CompilerParams(needs_layout_passes=False,
                                  use_tc_tiling_on_sc=False)


def _pre_body(e_ref, w12_ref, b_ref, wc_ref, r_ref, w3_ref,
              s12_ref, ew1_ref, ew2_ref, s3_ref):
    e = e_ref[...]
    bias = jnp.concatenate(
        [b_ref[...], jnp.zeros((1, 1), jnp.float32)], axis=1)
    s12_ref[...] = jnp.dot(e, w12_ref[...],
                           preferred_element_type=jnp.float32) + bias
    wc = wc_ref[...]
    ew1_ref[...] = jnp.dot(e, wc[:D, :], preferred_element_type=jnp.float32)
    ew2_ref[...] = jnp.dot(e, wc[D:, :], preferred_element_type=jnp.float32)

    @pl.when(pl.program_id(0) == 0)
    def _():
        s3_ref[...] = jnp.dot(r_ref[...], w3_ref[...],
                              preferred_element_type=jnp.float32)


def _precompute(entity_embedding, w12, b_att, W_conv, relation_embedding, w3):
    rb = 5000
    grid = N_ENT // rb
    return pl.pallas_call(
        _pre_body,
        grid=(grid,),
        in_specs=[
            pl.BlockSpec((rb, D), lambda i: (i, 0)),
            pl.BlockSpec((D, 2), lambda i: (0, 0)),
            pl.BlockSpec((1, 1), lambda i: (0, 0)),
            pl.BlockSpec((2 * D, D), lambda i: (0, 0)),
            pl.BlockSpec((N_REL, D), lambda i: (0, 0)),
            pl.BlockSpec((D, 1), lambda i: (0, 0)),
        ],
        out_specs=[
            pl.BlockSpec((rb, 2), lambda i: (i, 0)),
            pl.BlockSpec((rb, D), lambda i: (i, 0)),
            pl.BlockSpec((rb, D), lambda i: (i, 0)),
            pl.BlockSpec((N_REL, 1), lambda i: (0, 0)),
        ],
        out_shape=[
            jax.ShapeDtypeStruct((N_ENT, 2), jnp.float32),
            jax.ShapeDtypeStruct((N_ENT, D), jnp.float32),
            jax.ShapeDtypeStruct((N_ENT, D), jnp.float32),
            jax.ShapeDtypeStruct((N_REL, 1), jnp.float32),
        ],
    )(entity_embedding, w12, b_att.reshape(1, 1), W_conv,
      relation_embedding, w3)


def _attn_body(eids_hbm, adj_e_hbm, adj_r_hbm, s1_hbm, s2_hbm, s3_hbm,
               nbr_hbm, w_hbm, dnp_hbm,
               s2_v, s3_v, eids_v, s1_a, s1_b, nbr_a, nbr_b, rel_a, rel_b,
               nbrf_v, wf_v, dn_v, sem_a, sem_b):
    bl = eids_hbm.shape[0]
    per_w = bl // NW
    sp = CB * SB
    nsup = per_w // sp
    wid = lax.axis_index("s") * NC + lax.axis_index("c")
    base = wid * per_w

    pltpu.sync_copy(s2_hbm, s2_v)
    pltpu.sync_copy(s3_hbm, s3_v)
    pltpu.sync_copy(eids_hbm.at[pl.ds(base, per_w)], eids_v)
    zero = jnp.zeros((16,), jnp.float32)
    for i in range(L):
        dn_v[pl.ds(i * 16, 16)] = zero

    def issue_chunk(c, nbr_buf, rel_buf, s1_buf, sem):
        idx = eids_v.at[pl.ds(c * CB, CB)]
        pltpu.async_copy(adj_e_hbm.at[idx], nbr_buf, sem)
        pltpu.async_copy(adj_r_hbm.at[idx], rel_buf, sem)
        pltpu.async_copy(s1_hbm.at[idx], s1_buf, sem)

    def wait_chunk(c, nbr_buf, rel_buf, s1_buf, sem):
        idx = eids_v.at[pl.ds(c * CB, CB)]
        pltpu.make_async_copy(adj_e_hbm.at[idx], nbr_buf, sem).wait()
        pltpu.make_async_copy(adj_r_hbm.at[idx], rel_buf, sem).wait()
        pltpu.make_async_copy(s1_hbm.at[idx], s1_buf, sem).wait()

    def compute_chunk(sb, sc, nbr_buf, rel_buf, s1_buf):
        def per_group(jj, carry):
            s1g = s1_buf[pl.ds(jj * 16, 16)]
            for t in range(16):
                j = jj * 16 + t
                nrow = nbr_buf[j, :]
                rrow = rel_buf[j, :]
                s2g = plsc.load_gather(s2_v, [nrow])
                s3g = plsc.load_gather(s3_v, [rrow])
                logit = s2g + s3g + s1g[t]
                wv = jnp.exp(jnp.maximum(logit, 0.0))
                sj = sc * CB + j
                wf_v[pl.ds(sj * 16, 16)] = wv
                nbrf_v[pl.ds(sj * 16, 16)] = nrow
                li = lax.rem(sb + sj, L)
                cur = dn_v[pl.ds(li * 16, 16)]
                dn_v[pl.ds(li * 16, 16)] = cur + wv
            return carry

        lax.fori_loop(0, CB // 16, per_group, 0)

    def super_chunk(s, carry):
        sb = base + s * sp
        s0 = s * SB
        issue_chunk(s0, nbr_a, rel_a, s1_a, sem_a)

        def pair(h, c2):
            c0 = s0 + 2 * h
            issue_chunk(c0 + 1, nbr_b, rel_b, s1_b, sem_b)
            wait_chunk(c0, nbr_a, rel_a, s1_a, sem_a)
            compute_chunk(sb, 2 * h, nbr_a, rel_a, s1_a)

            @pl.when(2 * h + 2 < SB)
            def _():
                issue_chunk(c0 + 2, nbr_a, rel_a, s1_a, sem_a)

            wait_chunk(c0 + 1, nbr_b, rel_b, s1_b, sem_b)
            compute_chunk(sb, 2 * h + 1, nbr_b, rel_b, s1_b)
            return c2

        lax.fori_loop(0, SB // 2, pair, 0)
        pltpu.sync_copy(nbrf_v, nbr_hbm.at[pl.ds(sb * K, sp * K)])
        pltpu.sync_copy(wf_v, w_hbm.at[pl.ds(sb * K, sp * K)])
        return carry

    lax.fori_loop(0, nsup, super_chunk, 0)
    pltpu.sync_copy(dn_v, dnp_hbm.at[wid])


def _attention(eids_flat, adj_entity, adj_relation, s1, s2, s3):
    bl = eids_flat.shape[0]
    per_w = bl // NW
    sp = CB * SB
    mesh = plsc.VectorSubcoreMesh(core_axis_name="c", subcore_axis_name="s",
                                  num_cores=NC, num_subcores=NS)
    fn = functools.partial(
        pl.kernel,
        out_type=[
            jax.ShapeDtypeStruct((bl * K,), jnp.int32),
            jax.ShapeDtypeStruct((bl * K,), jnp.float32),
            jax.ShapeDtypeStruct((NW, L * K), jnp.float32),
        ],
        mesh=mesh,
        compiler_params=_SC_PARAMS,
        scratch_types=[
            pltpu.VMEM((N_ENT,), jnp.float32),
            pltpu.VMEM((N_REL,), jnp.float32),
            pltpu.VMEM((per_w,), jnp.int32),
            pltpu.VMEM((CB,), jnp.float32),
            pltpu.VMEM((CB,), jnp.float32),
            pltpu.VMEM((CB, K), jnp.int32),
            pltpu.VMEM((CB, K), jnp.int32),
            pltpu.VMEM((CB, K), jnp.int32),
            pltpu.VMEM((CB, K), jnp.int32),
            pltpu.VMEM((sp * K,), jnp.int32),
            pltpu.VMEM((sp * K,), jnp.float32),
            pltpu.VMEM((L * K,), jnp.float32),
            pltpu.SemaphoreType.DMA,
            pltpu.SemaphoreType.DMA,
        ],
    )(_attn_body)
    return fn(eids_flat, adj_entity, adj_relation, s1, s2, s3)


def _denom_body(p_ref, o_ref):
    o_ref[...] = 1.0 / jnp.sum(p_ref[...], axis=0, keepdims=True)


def _denominators(parts):
    return pl.pallas_call(
        _denom_body,
        out_shape=jax.ShapeDtypeStruct((1, L * K), jnp.float32),
    )(parts)


def _agg_body(eids_hbm, nbr_hbm, w_hbm, rd_hbm, ew1_hbm, ew2_hbm, bc_hbm,
              out_hbm,
              rd_v, bc_v, eid_v, idx_v, w_v, rows_a, rows_b, ent_sv, out_sv,
              sem_a, sem_b):
    bl = eids_hbm.shape[0]
    per_w = bl // NW
    wid = lax.axis_index("s") * NC + lax.axis_index("c")
    base = wid * per_w
    sp = CD * SCH
    nsup = per_w // sp
    ng = (CD * K) // 128

    pltpu.sync_copy(rd_hbm, rd_v)
    pltpu.sync_copy(bc_hbm, bc_v)
    bc0 = bc_v[pl.ds(0, 16)]
    bc1 = bc_v[pl.ds(16, 16)]
    bc2 = bc_v[pl.ds(32, 16)]
    bc3 = bc_v[pl.ds(48, 16)]

    def issue_chunk(c, rows_buf, sem):
        for g in range(ng):
            pltpu.async_copy(
                ew2_hbm.at[idx_v.at[pl.ds(c * CD * K + g * 128, 128)]],
                rows_buf.at[pl.ds(g * 128, 128)], sem)
        pltpu.async_copy(ew1_hbm.at[eid_v.at[pl.ds(c * CD, CD)]],
                         ent_sv.at[pl.ds(c * CD, CD)], sem)

    def wait_chunk(c, rows_buf, sem):
        for g in range(ng):
            pltpu.make_async_copy(
                ew2_hbm.at[idx_v.at[pl.ds(c * CD * K + g * 128, 128)]],
                rows_buf.at[pl.ds(g * 128, 128)], sem).wait()
        pltpu.make_async_copy(ew1_hbm.at[eid_v.at[pl.ds(c * CD, CD)]],
                              ent_sv.at[pl.ds(c * CD, CD)], sem).wait()

    def compute_chunk(sb, c, rows_buf):
        def per_j(j, carry):
            sj = c * CD + j
            li = lax.rem(sb + sj, L)
            av = w_v[pl.ds(sj * K, K)] * rd_v[pl.ds(li * K, K)]
            a0 = bc0 + ent_sv[sj, pl.ds(0, 16)]
            a1 = bc1 + ent_sv[sj, pl.ds(16, 16)]
            a2 = bc2 + ent_sv[sj, pl.ds(32, 16)]
            a3 = bc3 + ent_sv[sj, pl.ds(48, 16)]
            for k in range(K):
                r = j * K + k
                s = av[k]
                a0 = a0 + s * rows_buf[r, pl.ds(0, 16)]
                a1 = a1 + s * rows_buf[r, pl.ds(16, 16)]
                a2 = a2 + s * rows_buf[r, pl.ds(32, 16)]
                a3 = a3 + s * rows_buf[r, pl.ds(48, 16)]
            zero = jnp.zeros((16,), jnp.float32)
            out_sv[sj, pl.ds(0, 16)] = jnp.maximum(a0, zero)
            out_sv[sj, pl.ds(16, 16)] = jnp.maximum(a1, zero)
            out_sv[sj, pl.ds(32, 16)] = jnp.maximum(a2, zero)
            out_sv[sj, pl.ds(48, 16)] = jnp.maximum(a3, zero)
            return carry

        lax.fori_loop(0, CD, per_j, 0)

    def super_chunk(s, carry):
        sb = base + s * sp
        pltpu.sync_copy(eids_hbm.at[pl.ds(sb, sp)], eid_v)
        pltpu.sync_copy(nbr_hbm.at[pl.ds(sb * K, sp * K)], idx_v)
        pltpu.sync_copy(w_hbm.at[pl.ds(sb * K, sp * K)], w_v)
        issue_chunk(0, rows_a, sem_a)

        def pair(h, c2):
            c0 = 2 * h
            issue_chunk(c0 + 1, rows_b, sem_b)
            wait_chunk(c0, rows_a, sem_a)
            compute_chunk(sb, c0, rows_a)

            @pl.when(c0 + 2 < SCH)
            def _():
                issue_chunk(c0 + 2, rows_a, sem_a)

            wait_chunk(c0 + 1, rows_b, sem_b)
            compute_chunk(sb, c0 + 1, rows_b)
            return c2

        lax.fori_loop(0, SCH // 2, pair, 0)
        pltpu.sync_copy(out_sv, out_hbm.at[pl.ds(sb, sp)])
        return carry

    lax.fori_loop(0, nsup, super_chunk, 0)


def _aggregate(eids_flat, nbr_flat, w_flat, rdenom, ew1, ew2, b_conv):
    bl = eids_flat.shape[0]
    sp = CD * SCH
    mesh = plsc.VectorSubcoreMesh(core_axis_name="c", subcore_axis_name="s",
                                  num_cores=NC, num_subcores=NS)
    fn = functools.partial(
        pl.kernel,
        out_type=jax.ShapeDtypeStruct((bl, D), jnp.float32),
        mesh=mesh,
        compiler_params=_SC_PARAMS,
        scratch_types=[
            pltpu.VMEM((L * K,), jnp.float32),
            pltpu.VMEM((D,), jnp.float32),
            pltpu.VMEM((sp,), jnp.int32),
            pltpu.VMEM((sp * K,), jnp.int32),
            pltpu.VMEM((sp * K,), jnp.float32),
            pltpu.VMEM((CD * K, D), jnp.float32),
            pltpu.VMEM((CD * K, D), jnp.float32),
            pltpu.VMEM((sp, D), jnp.float32),
            pltpu.VMEM((sp, D), jnp.float32),
            pltpu.SemaphoreType.DMA,
            pltpu.SemaphoreType.DMA,
        ],
    )(_agg_body)
    return fn(eids_flat, nbr_flat, w_flat, rdenom, ew1, ew2, b_conv)


@jax.jit
def kernel(entity_ids, adj_entity, adj_relation, entity_embedding,
           relation_embedding, W_att, b_att, W_conv, b_conv):
    B, Ldim = entity_ids.shape
    bl = B * Ldim
    eids_flat = entity_ids.reshape(bl)

    w12 = jnp.concatenate([W_att[:D], W_att[D:2 * D]], axis=1)
    w3 = W_att[2 * D:]

    s12, ew1, ew2, s3m = _precompute(entity_embedding, w12, b_att, W_conv,
                                     relation_embedding, w3)
    s1 = s12[:, 0]
    s2 = s12[:, 1]
    s3 = s3m.reshape(N_REL)

    nbr_flat, w_flat, parts = _attention(
        eids_flat, adj_entity, adj_relation, s1, s2, s3)
    rdenom = _denominators(parts).reshape(L * K)
    out = _aggregate(eids_flat, nbr_flat, w_flat, rdenom, ew1, ew2, b_conv)
    return out.reshape(B, Ldim, D)
